# baseline (speedup 1.0000x reference)
import functools
import jax
import jax.numpy as jnp
from jax import lax
from jax.experimental import pallas as pl
from jax.experimental.pallas import tpu as pltpu
from jax.experimental.pallas import tpu_sc as plsc

NC = 2
NS = 16
L = 16
NW = NC * NS
KSEG = 1024
STRIP = 8192
UNROLL = 4
NPLN = 3


def _pool(B, C, HW):
  P = B * C
  PPW = P // NW
  NPASS = PPW // NPLN
  NSTRIP = HW // STRIP
  mesh = plsc.VectorSubcoreMesh(core_axis_name="c", subcore_axis_name="s")

  @functools.partial(
      pl.kernel,
      mesh=mesh,
      out_type=jax.ShapeDtypeStruct((P, KSEG), jnp.float32),
      compiler_params=pltpu.CompilerParams(
          needs_layout_passes=False, use_tc_tiling_on_sc=False
      ),
      scratch_types=[
          pltpu.VMEM((2, STRIP), jnp.int32),
          pltpu.VMEM((2, NPLN, STRIP), jnp.float32),
          pltpu.VMEM((L * KSEG,), jnp.float32),
          pltpu.VMEM((L * KSEG,), jnp.float32),
          pltpu.VMEM((L * KSEG,), jnp.float32),
          pltpu.VMEM((KSEG,), jnp.float32),
          pltpu.SemaphoreType.DMA,
          pltpu.SemaphoreType.DMA,
      ],
  )
  def k(img_hbm, spx_hbm, out_hbm, lbl_v, d_v, a0_v, a1_v, a2_v, row_v,
        sem0, sem1):
    wid = lax.axis_index("s") * NC + lax.axis_index("c")
    lane = lax.iota(jnp.int32, L)
    lane_k = lane * KSEG
    neg_inf = jnp.full((L,), -jnp.inf, jnp.float32)
    sems = (sem0, sem1)
    accs = (a0_v, a1_v, a2_v)

    def issue(s, slot, p0, b):
      off = s * STRIP
      pltpu.async_copy(
          spx_hbm.at[b, pl.ds(off, STRIP)], lbl_v.at[slot], sems[slot])
      pltpu.async_copy(
          img_hbm.at[pl.ds(p0, NPLN), pl.ds(off, STRIP)], d_v.at[slot],
          sems[slot])

    def wait(slot):
      pltpu.make_async_copy(
          spx_hbm.at[0, pl.ds(0, STRIP)], lbl_v.at[slot], sems[slot]).wait()
      pltpu.make_async_copy(
          img_hbm.at[pl.ds(0, NPLN), pl.ds(0, STRIP)], d_v.at[slot],
          sems[slot]).wait()

    for ps in range(NPASS):
      p0 = wid * PPW + NPLN * ps
      b = p0 // C

      def init_body(j, _):
        o = j * (4 * L)
        for u in range(4):
          a0_v[pl.ds(o + u * L, L)] = neg_inf
          a1_v[pl.ds(o + u * L, L)] = neg_inf
          a2_v[pl.ds(o + u * L, L)] = neg_inf
        return 0

      lax.fori_loop(0, KSEG // 4, init_body, 0)

      issue(0, 0, p0, b)

      def process(slot):
        def group_body(t, _):
          base = t * (UNROLL * L)
          for u in range(UNROLL):
            o = base + u * L
            lbl = lbl_v[slot, pl.ds(o, L)]
            idx = lane_k + lbl
            v0 = d_v[slot, 0, pl.ds(o, L)]
            v1 = d_v[slot, 1, pl.ds(o, L)]
            v2 = d_v[slot, 2, pl.ds(o, L)]
            c0 = plsc.load_gather(a0_v, [idx])
            c1 = plsc.load_gather(a1_v, [idx])
            c2 = plsc.load_gather(a2_v, [idx])
            plsc.store_scatter(a0_v, [idx], jnp.maximum(c0, v0))
            plsc.store_scatter(a1_v, [idx], jnp.maximum(c1, v1))
            plsc.store_scatter(a2_v, [idx], jnp.maximum(c2, v2))
          return 0

        lax.fori_loop(0, STRIP // (UNROLL * L), group_body, 0)

      def strip_body(s2, _):
        s = s2 * 2
        issue(s + 1, 1, p0, b)
        wait(0)
        process(0)

        @pl.when(s2 + 1 < NSTRIP // 2)
        def _():
          issue(s + 2, 0, p0, b)

        wait(1)
        process(1)
        return 0

      lax.fori_loop(0, NSTRIP // 2, strip_body, 0)

      for j in range(NPLN):
        acc = accs[j]

        def fin_body(jj, _):
          m = acc[pl.ds(jj * L, L)]
          for l in range(1, L):
            m = jnp.maximum(m, acc[pl.ds(l * KSEG + jj * L, L)])
          row_v[pl.ds(jj * L, L)] = m
          return 0

        lax.fori_loop(0, KSEG // L, fin_body, 0)
        pltpu.sync_copy(row_v, out_hbm.at[p0 + j])

  return k


@jax.jit
def kernel(img, spx):
  B, C, H, W = img.shape
  HW = H * W
  img2 = img.reshape(B * C, HW)
  spx2 = spx.reshape(B, HW).astype(jnp.int32)
  out = _pool(B, C, HW)(img2, spx2)
  return out.reshape(B, C, KSEG)
